# Initial kernel scaffold; baseline (speedup 1.0000x reference)
import jax
import jax.numpy as jnp
from jax.experimental import pallas as pl


def kernel(features, edge_index, adj_values, W1, b1, Wg0, bg0, Wg1, bg1, Wg2, bg2, Wh1, bh1, Wh2, bh2):
    raise NotImplementedError("write your pallas kernel here")


# speedup vs baseline: 7.0698x; 7.0698x over previous
import jax
import jax.numpy as jnp
from jax import lax
from jax.experimental import pallas as pl
from jax.experimental.pallas import tpu as pltpu
from jax.experimental.pallas import tpu_sc as plsc

N = 10000
E = 320000
F = 128
FH = F // 2
NCLASS = 64
NC = 2
NS = 16
EPT = E // NS
C = 80
NCH = EPT // C
WBT = 10
RPT = N // WBT
FV = FH // 16


def _spmm_body(h_hbm, srcr, dstr, adjr, out_hbm,
               src_all, dst_all, adj_all, rows, acc_sh, sem0, sem1):
    c = lax.axis_index("c")
    s = lax.axis_index("s")

    pltpu.sync_copy(srcr.at[s], src_all)
    pltpu.sync_copy(dstr.at[s], dst_all)
    pltpu.sync_copy(adjr.at[s], adj_all)

    coff = c * N

    def _off(g, carry):
        sl = pl.ds(g * 16, 16)
        src_all[sl] = src_all[sl] + coff
        return carry
    lax.fori_loop(0, EPT // 16, _off, 0)

    def _zrow(i, carry):
        for j in range(FV):
            rows[0, i, pl.ds(j * 16, 16)] = jnp.zeros((16,), jnp.float32)
        return carry
    lax.fori_loop(0, C, _zrow, 0)
    base_r = s * RPT

    @pl.when(s < WBT)
    def _zero_acc():
        for k in range(RPT // C):
            pltpu.sync_copy(rows.at[0], acc_sh.at[pl.ds(base_r + k * C, C)])
        rem = RPT % C
        if rem:
            pltpu.sync_copy(rows.at[0, pl.ds(0, rem)],
                            acc_sh.at[pl.ds(base_r + (RPT // C) * C, rem)])
    plsc.subcore_barrier()

    def g_start(i, b, sem):
        pltpu.async_copy(h_hbm.at[src_all.at[pl.ds(i * C, C)]],
                         rows.at[b], sem)

    def g_wait(b, sem):
        pltpu.make_async_copy(h_hbm.at[src_all.at[pl.ds(0, C)]],
                              rows.at[b], sem).wait()

    def compute(i, b):
        def _grp(g, carry):
            a16 = adj_all[pl.ds(i * C + g * 16, 16)]
            for el in range(16):
                aa = a16[el]
                e = g * 16 + el
                for j in range(FV):
                    sl = pl.ds(j * 16, 16)
                    rows[b, e, sl] = rows[b, e, sl] * aa
            return carry
        lax.fori_loop(0, C // 16, _grp, 0)

    def scat(i, b):
        pltpu.sync_copy(rows.at[b], acc_sh.at[dst_all.at[i]], add=True)

    g_start(0, 0, sem0)

    def _pair(k, carry):
        i0 = 2 * k
        g_start(i0 + 1, 1, sem1)
        g_wait(0, sem0)
        compute(i0, 0)
        scat(i0, 0)
        g_start(i0 + 2, 0, sem0)
        g_wait(1, sem1)
        compute(i0 + 1, 1)
        scat(i0 + 1, 1)
        return carry
    lax.fori_loop(0, NCH // 2 - 1, _pair, 0)
    g_start(NCH - 1, 1, sem1)
    g_wait(0, sem0)
    compute(NCH - 2, 0)
    scat(NCH - 2, 0)
    g_wait(1, sem1)
    compute(NCH - 1, 1)
    scat(NCH - 1, 1)

    plsc.subcore_barrier()

    @pl.when(s < WBT)
    def _writeback():
        pltpu.sync_copy(acc_sh.at[pl.ds(base_r, RPT)],
                        out_hbm.at[c, pl.ds(base_r, RPT)])


_spmm = pl.kernel(
    _spmm_body,
    out_type=jax.ShapeDtypeStruct((NC, N, FH), jnp.float32),
    mesh=plsc.VectorSubcoreMesh(core_axis_name="c", subcore_axis_name="s"),
    scratch_types=[
        pltpu.VMEM((EPT,), jnp.int32),
        pltpu.VMEM((NCH, C), jnp.int32),
        pltpu.VMEM((EPT,), jnp.float32),
        pltpu.VMEM((2, C, FH), jnp.float32),
        pltpu.VMEM_SHARED((N, FH), jnp.float32),
        pltpu.SemaphoreType.DMA,
        pltpu.SemaphoreType.DMA,
    ],
    compiler_params=pltpu.CompilerParams(use_tc_tiling_on_sc=False),
)


_BR = 1000


def _split(y):
    return y[:, :FH], y[:, FH:]


def _lin_relu_body(x_ref, w_ref, b_ref, o_ref):
    y = jnp.dot(x_ref[...], w_ref[...], preferred_element_type=jnp.float32)
    y = jnp.maximum(y + b_ref[...], 0.0)
    o_ref[0], o_ref[1] = _split(y)


def _lin_relu(x, w, b):
    return pl.pallas_call(
        _lin_relu_body,
        grid=(N // _BR,),
        in_specs=[
            pl.BlockSpec((_BR, F), lambda i: (i, 0)),
            pl.BlockSpec((F, F), lambda i: (0, 0)),
            pl.BlockSpec((1, F), lambda i: (0, 0)),
        ],
        out_specs=pl.BlockSpec((NC, _BR, FH), lambda i: (0, i, 0)),
        out_shape=jax.ShapeDtypeStruct((NC, N, FH), jnp.float32),
    )(x, w, b.reshape(1, F))


def _lin_relu2_body(p_ref, w_ref, b_ref, o_ref):
    y = (jnp.dot(p_ref[0], w_ref[:FH], preferred_element_type=jnp.float32)
         + jnp.dot(p_ref[1], w_ref[FH:], preferred_element_type=jnp.float32))
    y = jnp.maximum(y + b_ref[...], 0.0)
    o_ref[0], o_ref[1] = _split(y)


def _lin_relu2(p, w, b):
    return pl.pallas_call(
        _lin_relu2_body,
        grid=(N // _BR,),
        in_specs=[
            pl.BlockSpec((NC, _BR, FH), lambda i: (0, i, 0)),
            pl.BlockSpec((F, F), lambda i: (0, 0)),
            pl.BlockSpec((1, F), lambda i: (0, 0)),
        ],
        out_specs=pl.BlockSpec((NC, _BR, FH), lambda i: (0, i, 0)),
        out_shape=jax.ShapeDtypeStruct((NC, N, FH), jnp.float32),
    )(p, w, b.reshape(1, F))


def _head_body(p_ref, wg_ref, bg_ref, w1_ref, b1_ref, w2_ref, b2_ref, o_ref):
    h = (jnp.dot(p_ref[0], wg_ref[:FH], preferred_element_type=jnp.float32)
         + jnp.dot(p_ref[1], wg_ref[FH:], preferred_element_type=jnp.float32))
    h = jnp.maximum(h + bg_ref[...], 0.0)
    t = jnp.dot(h, w1_ref[...], preferred_element_type=jnp.float32)
    t = jnp.maximum(t + b1_ref[...], 0.0)
    o_ref[...] = (
        jnp.dot(t, w2_ref[...], preferred_element_type=jnp.float32)
        + b2_ref[...]
    )


def _head(p, wg, bg, w1, b1, w2, b2):
    return pl.pallas_call(
        _head_body,
        grid=(N // _BR,),
        in_specs=[
            pl.BlockSpec((NC, _BR, FH), lambda i: (0, i, 0)),
            pl.BlockSpec((F, F), lambda i: (0, 0)),
            pl.BlockSpec((1, F), lambda i: (0, 0)),
            pl.BlockSpec((F, F), lambda i: (0, 0)),
            pl.BlockSpec((1, F), lambda i: (0, 0)),
            pl.BlockSpec((F, NCLASS), lambda i: (0, 0)),
            pl.BlockSpec((1, NCLASS), lambda i: (0, 0)),
        ],
        out_specs=pl.BlockSpec((_BR, NCLASS), lambda i: (i, 0)),
        out_shape=jax.ShapeDtypeStruct((N, NCLASS), jnp.float32),
    )(p, wg, bg.reshape(1, F), w1, b1.reshape(1, F), w2,
      b2.reshape(1, NCLASS))


@jax.jit
def kernel(features, edge_index, adj_values,
           W1, b1, Wg0, bg0, Wg1, bg1, Wg2, bg2, Wh1, bh1, Wh2, bh2):
    srcr = edge_index[1].reshape(NS, EPT)
    dstr = edge_index[0].reshape(NS, NCH, C)
    adjr = adj_values.reshape(NS, EPT)

    h = _lin_relu(features, W1, b1)
    p = _spmm(h.reshape(NC * N, FH), srcr, dstr, adjr)
    h = _lin_relu2(p, Wg0, bg0)
    p = _spmm(h.reshape(NC * N, FH), srcr, dstr, adjr)
    h = _lin_relu2(p, Wg1, bg1)
    p = _spmm(h.reshape(NC * N, FH), srcr, dstr, adjr)
    out = _head(p, Wg2, bg2, Wh1, bh1, Wh2, bh2)
    return (out,)
